# baseline (speedup 1.0000x reference)
import functools

import jax
import jax.numpy as jnp
from jax import lax
from jax.experimental import pallas as pl
from jax.experimental.pallas import tpu as pltpu
from jax.experimental.pallas import tpu_sc as plsc

_VOCAB = 1000000
_DIM = 64
_BATCH = 4096
_SEQ = 200
_ALPHA = 0.1

_info = plsc.get_sparse_core_info()
_NC, _NS, _L = _info.num_cores, _info.num_subcores, _info.num_lanes
_NW = _NC * _NS

_ROWS = _BATCH * _SEQ
_RPW = _ROWS // _NW
_CHUNK = 128
_NCHUNK = _RPW // _CHUNK
_VPR = _DIM // _L


def _sc_body(ids_hbm, table_hbm, out_hbm, idx_v, in0, in1, out0, out1,
             sem_in0, sem_in1, sem_out0, sem_out1):
    wid = lax.axis_index("s") * _NC + lax.axis_index("c")
    base = wid * _RPW

    inbufs = (in0, in1)
    outbufs = (out0, out1)
    sem_ins = (sem_in0, sem_in1)
    sem_outs = (sem_out0, sem_out1)

    pltpu.sync_copy(ids_hbm.at[wid], idx_v)

    for q in range(_VPR):
        in0[0, pl.ds(q * _L, _L)] = jnp.zeros((_L,), jnp.float32)

    pltpu.make_async_copy(
        table_hbm.at[idx_v.at[0]], in0.at[pl.ds(1, _CHUNK)], sem_in0
    ).start()

    def outer(i, carry):
        for b in range(2):
            c = i * 2 + b
            ib, ob = inbufs[b], outbufs[b]
            nib = inbufs[1 - b]

            pltpu.make_async_copy(
                table_hbm.at[idx_v.at[c]], ib.at[pl.ds(1, _CHUNK)], sem_ins[b]
            ).wait()

            @pl.when(c + 1 < _NCHUNK)
            def _():
                pltpu.make_async_copy(
                    table_hbm.at[idx_v.at[c + 1]],
                    nib.at[pl.ds(1, _CHUNK)], sem_ins[1 - b]
                ).start()

            for q in range(_VPR):
                nib[0, pl.ds(q * _L, _L)] = ib[_CHUNK, pl.ds(q * _L, _L)]

            @pl.when(c >= 2)
            def _():
                pltpu.make_async_copy(
                    ob, out_hbm.at[pl.ds(base + (c - 2) * _CHUNK, _CHUNK)],
                    sem_outs[b]
                ).wait()

            def row_body(r, _):
                g = c * _CHUNK + r
                f = lax.select(g % _SEQ == 0, jnp.float32(0.0),
                               jnp.float32(_ALPHA))
                for q in range(_VPR):
                    cur = ib[r + 1, pl.ds(q * _L, _L)]
                    prev = ib[r, pl.ds(q * _L, _L)]
                    ob[r, pl.ds(q * _L, _L)] = cur * (1.0 + f) - prev * f
                return 0

            lax.fori_loop(0, _CHUNK, row_body, 0)

            pltpu.make_async_copy(
                ob, out_hbm.at[pl.ds(base + c * _CHUNK, _CHUNK)], sem_outs[b]
            ).start()
        return carry

    lax.fori_loop(0, _NCHUNK // 2, outer, 0)

    for b in range(2):
        c = _NCHUNK - 2 + b
        pltpu.make_async_copy(
            outbufs[b], out_hbm.at[pl.ds(base + c * _CHUNK, _CHUNK)],
            sem_outs[b]
        ).wait()


@jax.jit
def _gather_enhance(ids, table):
    mesh = plsc.VectorSubcoreMesh(core_axis_name="c", subcore_axis_name="s")
    run = functools.partial(
        pl.kernel,
        mesh=mesh,
        compiler_params=pltpu.CompilerParams(use_tc_tiling_on_sc=False),
        out_type=jax.ShapeDtypeStruct((_ROWS, _DIM), jnp.float32),
        scratch_types=[
            pltpu.VMEM((_NCHUNK, _CHUNK), jnp.int32),
            pltpu.VMEM((_CHUNK + 1, _DIM), jnp.float32),
            pltpu.VMEM((_CHUNK + 1, _DIM), jnp.float32),
            pltpu.VMEM((_CHUNK, _DIM), jnp.float32),
            pltpu.VMEM((_CHUNK, _DIM), jnp.float32),
            pltpu.SemaphoreType.DMA,
            pltpu.SemaphoreType.DMA,
            pltpu.SemaphoreType.DMA,
            pltpu.SemaphoreType.DMA,
        ],
    )(_sc_body)
    return run(ids, table)


def kernel(input_ids, table):
    ids = input_ids.reshape(_NW, _NCHUNK, _CHUNK)
    out = _gather_enhance(ids, table)
    return out.reshape(_BATCH, _SEQ, _DIM)
